# baseline (speedup 1.0000x reference)
import functools

import jax
import jax.numpy as jnp
from jax import lax
from jax.experimental import pallas as pl
from jax.experimental.pallas import tpu as pltpu
from jax.experimental.pallas import tpu_sc as plsc

T = 2048
D = 768
E = 8
H = 1536
B = 256
S = 6144
NB = S // B
NBP = 128

_f32 = jnp.float32
_i32 = jnp.int32


def _router_body(hs_ref, rw_ref, pos_ref, w0_ref, w1_ref, be_ref, nb_ref):
    hs = hs_ref[...]
    rw = rw_ref[...]
    st = lax.dot_general(rw, hs, (((1,), (1,)), ((), ())),
                         preferred_element_type=_f32)
    eio = lax.broadcasted_iota(_i32, (E, T), 0)
    m0 = jnp.max(st, axis=0, keepdims=True)
    idx0 = jnp.min(jnp.where(st == m0, eio, E), axis=0, keepdims=True)
    neg = jnp.where(eio == idx0, -jnp.inf, st)
    m1 = jnp.max(neg, axis=0, keepdims=True)
    idx1 = jnp.min(jnp.where(neg == m1, eio, E), axis=0, keepdims=True)

    sn = lax.dot_general(hs, rw, (((1,), (1,)), ((), ())),
                         preferred_element_type=_f32)
    lio = lax.broadcasted_iota(_i32, (T, E), 1)
    m0n = jnp.max(sn, axis=1, keepdims=True)
    i0n = jnp.min(jnp.where(sn == m0n, lio, E), axis=1, keepdims=True)
    m1n = jnp.max(jnp.where(lio == i0n, -jnp.inf, sn), axis=1, keepdims=True)
    sx = jnp.exp(m1n - m0n)
    w0_ref[...] = jnp.broadcast_to(1.0 / (1.0 + sx), (T, 16))
    w1_ref[...] = jnp.broadcast_to(sx / (1.0 + sx), (T, 16))

    erow = jnp.concatenate([idx0, idx1], axis=1)
    pio = lax.broadcasted_iota(_i32, (E, 2 * T), 0)
    p2 = (pio == erow).astype(_f32)
    counts = jnp.sum(p2, axis=1, keepdims=True)

    lane = lax.broadcasted_iota(_i32, (E, 2 * T), 1)
    c2 = p2
    sh = 1
    while sh < 2 * T:
        c2 = c2 + jnp.where(lane >= sh, pltpu.roll(c2, sh, 1), 0.0)
        sh *= 2
    rank = jnp.sum(p2 * c2, axis=0, keepdims=True) - 1.0

    pc = jnp.floor((counts + (B - 1)) / B) * B
    ts = (lax.broadcasted_iota(_i32, (E, E), 0)
          > lax.broadcasted_iota(_i32, (E, E), 1)).astype(_f32)
    offm = lax.dot_general(ts, jnp.broadcast_to(pc, (E, NBP)),
                           (((1,), (0,)), ((), ())),
                           preferred_element_type=_f32)
    off_pad = offm[:, :1]
    off_tok = jnp.sum(p2 * off_pad, axis=0, keepdims=True)
    pos_ref[...] = jnp.reshape(off_tok + rank, (2 * T,)).astype(_i32)

    nblk = pc / B
    cume = lax.dot_general(ts, jnp.broadcast_to(nblk, (E, NBP)),
                           (((1,), (0,)), ((), ())),
                           preferred_element_type=_f32)
    bio = lax.broadcasted_iota(_i32, (E, NBP), 1).astype(_f32)
    be = jnp.sum((bio >= cume).astype(_f32), axis=0) - 1.0
    be_ref[...] = be.astype(_i32)
    nb_ref[...] = jnp.broadcast_to(jnp.sum(nblk), (8,)).astype(_i32)


_router = pl.pallas_call(
    _router_body,
    out_shape=[
        jax.ShapeDtypeStruct((2 * T,), _i32),
        jax.ShapeDtypeStruct((T, 16), _f32),
        jax.ShapeDtypeStruct((T, 16), _f32),
        jax.ShapeDtypeStruct((NBP,), _i32),
        jax.ShapeDtypeStruct((8,), _i32),
    ],
)


def _dispatch_body(x_hbm, pos_hbm, xs_hbm, idx_v, rows_v, sem):
    c = lax.axis_index("c")
    s = lax.axis_index("s")
    pltpu.sync_copy(pos_hbm.at[pl.ds(c * T + s * 128, 128)], idx_v)
    pltpu.sync_copy(x_hbm.at[pl.ds(s * 128, 128)], rows_v)
    pltpu.async_copy(rows_v, xs_hbm.at[idx_v], sem).wait()


@functools.cache
def _make_dispatch():
    return pl.kernel(
        _dispatch_body,
        out_type=jax.ShapeDtypeStruct((S, D), _f32),
        mesh=plsc.VectorSubcoreMesh(core_axis_name="c", subcore_axis_name="s"),
        scratch_types=[
            pltpu.VMEM((128,), _i32),
            pltpu.VMEM((128, D), _f32),
            pltpu.SemaphoreType.DMA,
        ],
    )


NBMAX = S // B - 1


def _ffn_body(be_s, nb_s, xs_h, wg_h, wu_h, wd_h, ys_h,
              xv, gv, uv, dv, yv, sx0, sx1, sw0, sw1, sy0, sy1):
    sx = (sx0, sx1)
    sy = (sy0, sy1)
    nbocc = nb_s[0]

    def start_x(b, par):
        pltpu.make_async_copy(xs_h.at[pl.ds(b * B, B)], xv.at[par],
                              sx[par]).start()

    def start_w(e, widx_is0, cond):
        for wsel, wcond in ((0, cond & widx_is0), (1, cond & (~widx_is0))):
            @pl.when(wcond)
            def _():
                pltpu.make_async_copy(wg_h.at[e], gv.at[wsel],
                                      (sw0, sw1)[wsel]).start()
                pltpu.make_async_copy(wu_h.at[e], uv.at[wsel],
                                      (sw0, sw1)[wsel]).start()
                pltpu.make_async_copy(wd_h.at[e], dv.at[wsel],
                                      (sw0, sw1)[wsel]).start()

    def wait_w(widx_is0, cond):
        for wsel, wcond in ((0, cond & widx_is0), (1, cond & (~widx_is0))):
            @pl.when(wcond)
            def _():
                s = (sw0, sw1)[wsel]
                pltpu.make_async_copy(wg_h.at[0], gv.at[wsel], s).wait()
                pltpu.make_async_copy(wu_h.at[0], uv.at[wsel], s).wait()
                pltpu.make_async_copy(wd_h.at[0], dv.at[wsel], s).wait()

    e0 = be_s[0]
    start_x(0, 0)
    start_w(e0, jnp.bool_(True), jnp.bool_(True))
    widx = jnp.int32(0)
    cur_e = e0
    fresh = jnp.bool_(True)

    for b in range(NBMAX):
        par = b % 2
        live = b < nbocc
        has_next = (b + 1) < nbocc
        nxt_e = be_s[min(b + 1, NBMAX - 1)]
        need_w = has_next & (nxt_e != cur_e)
        nwidx = jnp.where(need_w, 1 - widx, widx)

        @pl.when(has_next)
        def _(b=b):
            start_x(b + 1, (b + 1) % 2)

        start_w(nxt_e, nwidx == 0, need_w)

        @pl.when(live)
        def _():
            pltpu.make_async_copy(xs_h.at[pl.ds(0, B)], xv.at[par],
                                  sx[par]).wait()

        wait_w(widx == 0, live & fresh)

        if b >= 2:
            @pl.when(live)
            def _():
                pltpu.make_async_copy(yv.at[par], ys_h.at[pl.ds(0, B)],
                                      sy[par]).wait()

        @pl.when(live)
        def _():
            x = xv[par]
            g = lax.dot_general(x, gv[widx], (((1,), (1,)), ((), ())),
                                preferred_element_type=_f32)
            u = lax.dot_general(x, uv[widx], (((1,), (1,)), ((), ())),
                                preferred_element_type=_f32)
            h = g * lax.logistic(g) * u
            yv[par] = lax.dot_general(h, dv[widx], (((1,), (1,)), ((), ())),
                                      preferred_element_type=_f32)

        @pl.when(live)
        def _(b=b):
            pltpu.make_async_copy(yv.at[par], ys_h.at[pl.ds(b * B, B)],
                                  sy[par]).start()

        widx, cur_e, fresh = nwidx, nxt_e, need_w

    for par in (0, 1):
        pltpu.make_async_copy(yv.at[par], ys_h.at[pl.ds(0, B)],
                              sy[par]).wait()


_ffn = pl.pallas_call(
    _ffn_body,
    in_specs=[
        pl.BlockSpec(memory_space=pltpu.SMEM),
        pl.BlockSpec(memory_space=pltpu.SMEM),
        pl.BlockSpec(memory_space=pl.ANY),
        pl.BlockSpec(memory_space=pl.ANY),
        pl.BlockSpec(memory_space=pl.ANY),
        pl.BlockSpec(memory_space=pl.ANY),
    ],
    out_specs=pl.BlockSpec(memory_space=pl.ANY),
    out_shape=jax.ShapeDtypeStruct((S, D), _f32),
    scratch_shapes=[
        pltpu.VMEM((2, B, D), _f32),
        pltpu.VMEM((2, H, D), _f32),
        pltpu.VMEM((2, H, D), _f32),
        pltpu.VMEM((2, D, H), _f32),
        pltpu.VMEM((2, B, D), _f32),
        pltpu.SemaphoreType.DMA,
        pltpu.SemaphoreType.DMA,
        pltpu.SemaphoreType.DMA,
        pltpu.SemaphoreType.DMA,
        pltpu.SemaphoreType.DMA,
        pltpu.SemaphoreType.DMA,
    ],
)


def _combine_body(pos_hbm, w0_hbm, w1_hbm, ys_hbm, out_hbm,
                  i0_v, i1_v, w0_v, w1_v, z0_v, z1_v, s0, s1):
    c = lax.axis_index("c")
    s = lax.axis_index("s")
    tb = (c * 16 + s) * 64
    pltpu.sync_copy(pos_hbm.at[pl.ds(tb, 64)], i0_v)
    pltpu.sync_copy(pos_hbm.at[pl.ds(T + tb, 64)], i1_v)
    cp0 = pltpu.async_copy(ys_hbm.at[i0_v], z0_v, s0)
    cp1 = pltpu.async_copy(ys_hbm.at[i1_v], z1_v, s1)
    pltpu.sync_copy(w0_hbm.at[pl.ds(tb, 64)], w0_v)
    pltpu.sync_copy(w1_hbm.at[pl.ds(tb, 64)], w1_v)
    cp0.wait()
    cp1.wait()

    def row(r, _):
        a = w0_v[r, pl.ds(0, 16)]
        b = w1_v[r, pl.ds(0, 16)]
        for ch in range(D // 16):
            sl = pl.ds(16 * ch, 16)
            z0_v[r, sl] = a * z0_v[r, sl] + b * z1_v[r, sl]
        return 0

    lax.fori_loop(0, 64, row, 0)
    pltpu.sync_copy(z0_v, out_hbm.at[pl.ds(tb, 64)])


@functools.cache
def _make_combine():
    return pl.kernel(
        _combine_body,
        out_type=jax.ShapeDtypeStruct((T, D), _f32),
        mesh=plsc.VectorSubcoreMesh(core_axis_name="c", subcore_axis_name="s"),
        scratch_types=[
            pltpu.VMEM((64,), _i32),
            pltpu.VMEM((64,), _i32),
            pltpu.VMEM((64, 16), _f32),
            pltpu.VMEM((64, 16), _f32),
            pltpu.VMEM((64, D), _f32),
            pltpu.VMEM((64, D), _f32),
            pltpu.SemaphoreType.DMA,
            pltpu.SemaphoreType.DMA,
        ],
    )


def kernel(x, router_w, w_gate, w_up, w_down):
    b, seq, d = x.shape
    hs = x.reshape(T, D)
    pos, w0, w1, be, nb = _router(hs, router_w)
    xs = _make_dispatch()(hs, pos)
    ys = _ffn(be, nb, xs, w_gate, w_up, w_down)
    out = _make_combine()(pos, w0, w1, ys)
    return out.reshape(b, seq, d)


# speedup vs baseline: 1.1618x; 1.1618x over previous
import functools

import jax
import jax.numpy as jnp
from jax import lax
from jax.experimental import pallas as pl
from jax.experimental.pallas import tpu as pltpu
from jax.experimental.pallas import tpu_sc as plsc

T = 2048
D = 768
E = 8
H = 1536
B = 256
S = 6144
NB = S // B
NBP = 128

_f32 = jnp.float32
_i32 = jnp.int32


def _router_body(hs_ref, rw_ref, pos_ref, w0_ref, w1_ref, be_ref, nb_ref):
    hs = hs_ref[...]
    rw = rw_ref[...]
    st = lax.dot_general(rw, hs, (((1,), (1,)), ((), ())),
                         preferred_element_type=_f32)
    eio = lax.broadcasted_iota(_i32, (E, T), 0)
    m0 = jnp.max(st, axis=0, keepdims=True)
    idx0 = jnp.min(jnp.where(st == m0, eio, E), axis=0, keepdims=True)
    neg = jnp.where(eio == idx0, -jnp.inf, st)
    m1 = jnp.max(neg, axis=0, keepdims=True)
    idx1 = jnp.min(jnp.where(neg == m1, eio, E), axis=0, keepdims=True)

    sn = lax.dot_general(hs, rw, (((1,), (1,)), ((), ())),
                         preferred_element_type=_f32)
    lio = lax.broadcasted_iota(_i32, (T, E), 1)
    m0n = jnp.max(sn, axis=1, keepdims=True)
    i0n = jnp.min(jnp.where(sn == m0n, lio, E), axis=1, keepdims=True)
    m1n = jnp.max(jnp.where(lio == i0n, -jnp.inf, sn), axis=1, keepdims=True)
    sx = jnp.exp(m1n - m0n)
    w0_ref[...] = jnp.broadcast_to(1.0 / (1.0 + sx), (T, 16))
    w1_ref[...] = jnp.broadcast_to(sx / (1.0 + sx), (T, 16))

    erow = jnp.concatenate([idx0, idx1], axis=1)
    pio = lax.broadcasted_iota(_i32, (E, 2 * T), 0)
    p2 = (pio == erow).astype(_f32)
    counts = jnp.sum(p2, axis=1, keepdims=True)

    lane = lax.broadcasted_iota(_i32, (E, 2 * T), 1)
    c2 = p2
    sh = 1
    while sh < 2 * T:
        c2 = c2 + jnp.where(lane >= sh, pltpu.roll(c2, sh, 1), 0.0)
        sh *= 2
    rank = jnp.sum(p2 * c2, axis=0, keepdims=True) - 1.0

    pc = jnp.floor((counts + (B - 1)) / B) * B
    ts = (lax.broadcasted_iota(_i32, (E, E), 0)
          > lax.broadcasted_iota(_i32, (E, E), 1)).astype(_f32)
    offm = lax.dot_general(ts, jnp.broadcast_to(pc, (E, NBP)),
                           (((1,), (0,)), ((), ())),
                           preferred_element_type=_f32)
    off_pad = offm[:, :1]
    off_tok = jnp.sum(p2 * off_pad, axis=0, keepdims=True)
    pos_ref[...] = jnp.reshape(off_tok + rank, (2 * T,)).astype(_i32)

    nblk = pc / B
    cume = lax.dot_general(ts, jnp.broadcast_to(nblk, (E, NBP)),
                           (((1,), (0,)), ((), ())),
                           preferred_element_type=_f32)
    bio = lax.broadcasted_iota(_i32, (E, NBP), 1).astype(_f32)
    be = jnp.sum((bio >= cume).astype(_f32), axis=0) - 1.0
    be_ref[...] = be.astype(_i32)
    nb_ref[...] = jnp.broadcast_to(jnp.sum(nblk), (8,)).astype(_i32)


_router = pl.pallas_call(
    _router_body,
    out_shape=[
        jax.ShapeDtypeStruct((2 * T,), _i32),
        jax.ShapeDtypeStruct((T, 16), _f32),
        jax.ShapeDtypeStruct((T, 16), _f32),
        jax.ShapeDtypeStruct((NBP,), _i32),
        jax.ShapeDtypeStruct((8,), _i32),
    ],
)


def _dispatch_body(x_hbm, pos_hbm, xs_hbm, idx_v, rows_v, sem):
    c = lax.axis_index("c")
    s = lax.axis_index("s")
    pltpu.sync_copy(pos_hbm.at[pl.ds(c * T + s * 128, 128)], idx_v)
    pltpu.sync_copy(x_hbm.at[pl.ds(s * 128, 128)], rows_v)
    pltpu.async_copy(rows_v, xs_hbm.at[idx_v], sem).wait()


@functools.cache
def _make_dispatch():
    return pl.kernel(
        _dispatch_body,
        out_type=jax.ShapeDtypeStruct((S, D), _f32),
        mesh=plsc.VectorSubcoreMesh(core_axis_name="c", subcore_axis_name="s"),
        scratch_types=[
            pltpu.VMEM((128,), _i32),
            pltpu.VMEM((128, D), _f32),
            pltpu.SemaphoreType.DMA,
        ],
    )


NBMAX = S // B - 1


def _ffn_body(be_s, nb_s, xs_h, wg_h, wu_h, wd_h, ys_h,
              xv, gv, uv, dv, yv, sx0, sx1, swA, swB, swC, sy0, sy1):
    sx = (sx0, sx1)
    sy = (sy0, sy1)
    sw = (swA, swB, swC)
    nbocc = nb_s[0]

    def start_x(b, par):
        pltpu.make_async_copy(xs_h.at[pl.ds(b * B, B)], xv.at[par],
                              sx[par]).start()

    def start_w(e, widx, cond):
        for wsel in range(3):
            @pl.when(cond & (widx == wsel))
            def _(wsel=wsel):
                pltpu.make_async_copy(wg_h.at[e], gv.at[wsel], sw[wsel]).start()
                pltpu.make_async_copy(wu_h.at[e], uv.at[wsel], sw[wsel]).start()
                pltpu.make_async_copy(wd_h.at[e], dv.at[wsel], sw[wsel]).start()

    def wait_w(widx, cond):
        for wsel in range(3):
            @pl.when(cond & (widx == wsel))
            def _(wsel=wsel):
                pltpu.make_async_copy(wg_h.at[0], gv.at[wsel], sw[wsel]).wait()
                pltpu.make_async_copy(wu_h.at[0], uv.at[wsel], sw[wsel]).wait()
                pltpu.make_async_copy(wd_h.at[0], dv.at[wsel], sw[wsel]).wait()

    true_ = jnp.bool_(True)
    e0 = be_s[0]
    e1p = be_s[1]
    start_x(0, 0)
    start_w(e0, jnp.int32(0), true_)
    new1p = (1 < nbocc) & (e1p != e0)
    start_w(e1p, jnp.int32(1), new1p)
    widx = jnp.int32(0)
    cur_e = e0
    fresh = true_

    for b in range(NBMAX):
        par = b % 2
        live = b < nbocc
        has_next = (b + 1) < nbocc
        e1 = be_s[min(b + 1, NBMAX - 1)]
        e2 = be_s[min(b + 2, NBMAX - 1)]
        new1 = has_next & (e1 != cur_e)
        new2 = ((b + 2) < nbocc) & (e2 != e1)
        w1 = (widx + new1.astype(_i32)) % 3
        w2 = (w1 + new2.astype(_i32)) % 3

        @pl.when(has_next)
        def _(b=b):
            start_x(b + 1, (b + 1) % 2)

        start_w(e2, w2, new2)

        @pl.when(live)
        def _():
            pltpu.make_async_copy(xs_h.at[pl.ds(0, B)], xv.at[par],
                                  sx[par]).wait()

        wait_w(widx, live & fresh)

        if b >= 2:
            @pl.when(live)
            def _():
                pltpu.make_async_copy(yv.at[par], ys_h.at[pl.ds(0, B)],
                                      sy[par]).wait()

        @pl.when(live)
        def _():
            x = xv[par]
            g = lax.dot_general(x, gv[widx], (((1,), (1,)), ((), ())),
                                preferred_element_type=_f32)
            u = lax.dot_general(x, uv[widx], (((1,), (1,)), ((), ())),
                                preferred_element_type=_f32)
            h = g * lax.logistic(g) * u
            yv[par] = lax.dot_general(h, dv[widx], (((1,), (1,)), ((), ())),
                                      preferred_element_type=_f32)

        @pl.when(live)
        def _(b=b):
            pltpu.make_async_copy(yv.at[par], ys_h.at[pl.ds(b * B, B)],
                                  sy[par]).start()

        widx, cur_e, fresh = w1, e1, new1

    for par in (0, 1):
        pltpu.make_async_copy(yv.at[par], ys_h.at[pl.ds(0, B)],
                              sy[par]).wait()


_ffn = pl.pallas_call(
    _ffn_body,
    in_specs=[
        pl.BlockSpec(memory_space=pltpu.SMEM),
        pl.BlockSpec(memory_space=pltpu.SMEM),
        pl.BlockSpec(memory_space=pl.ANY),
        pl.BlockSpec(memory_space=pl.ANY),
        pl.BlockSpec(memory_space=pl.ANY),
        pl.BlockSpec(memory_space=pl.ANY),
    ],
    out_specs=pl.BlockSpec(memory_space=pl.ANY),
    out_shape=jax.ShapeDtypeStruct((S, D), _f32),
    scratch_shapes=[
        pltpu.VMEM((2, B, D), _f32),
        pltpu.VMEM((3, H, D), _f32),
        pltpu.VMEM((3, H, D), _f32),
        pltpu.VMEM((3, D, H), _f32),
        pltpu.VMEM((2, B, D), _f32),
        pltpu.SemaphoreType.DMA,
        pltpu.SemaphoreType.DMA,
        pltpu.SemaphoreType.DMA,
        pltpu.SemaphoreType.DMA,
        pltpu.SemaphoreType.DMA,
        pltpu.SemaphoreType.DMA,
        pltpu.SemaphoreType.DMA,
    ],
)


def _combine_body(pos_hbm, w0_hbm, w1_hbm, ys_hbm, out_hbm,
                  i0_v, i1_v, w0_v, w1_v, z0_v, z1_v, s0, s1):
    c = lax.axis_index("c")
    s = lax.axis_index("s")
    tb = (c * 16 + s) * 64
    pltpu.sync_copy(pos_hbm.at[pl.ds(tb, 64)], i0_v)
    pltpu.sync_copy(pos_hbm.at[pl.ds(T + tb, 64)], i1_v)
    cp0 = pltpu.async_copy(ys_hbm.at[i0_v], z0_v, s0)
    cp1 = pltpu.async_copy(ys_hbm.at[i1_v], z1_v, s1)
    pltpu.sync_copy(w0_hbm.at[pl.ds(tb, 64)], w0_v)
    pltpu.sync_copy(w1_hbm.at[pl.ds(tb, 64)], w1_v)
    cp0.wait()
    cp1.wait()

    def row(r, _):
        a = w0_v[r, pl.ds(0, 16)]
        b = w1_v[r, pl.ds(0, 16)]
        for ch in range(D // 16):
            sl = pl.ds(16 * ch, 16)
            z0_v[r, sl] = a * z0_v[r, sl] + b * z1_v[r, sl]
        return 0

    lax.fori_loop(0, 64, row, 0)
    pltpu.sync_copy(z0_v, out_hbm.at[pl.ds(tb, 64)])


@functools.cache
def _make_combine():
    return pl.kernel(
        _combine_body,
        out_type=jax.ShapeDtypeStruct((T, D), _f32),
        mesh=plsc.VectorSubcoreMesh(core_axis_name="c", subcore_axis_name="s"),
        scratch_types=[
            pltpu.VMEM((64,), _i32),
            pltpu.VMEM((64,), _i32),
            pltpu.VMEM((64, 16), _f32),
            pltpu.VMEM((64, 16), _f32),
            pltpu.VMEM((64, D), _f32),
            pltpu.VMEM((64, D), _f32),
            pltpu.SemaphoreType.DMA,
            pltpu.SemaphoreType.DMA,
        ],
    )


def kernel(x, router_w, w_gate, w_up, w_down):
    b, seq, d = x.shape
    hs = x.reshape(T, D)
    pos, w0, w1, be, nb = _router(hs, router_w)
    xs = _make_dispatch()(hs, pos)
    ys = _ffn(be, nb, xs, w_gate, w_up, w_down)
    out = _make_combine()(pos, w0, w1, ys)
    return out.reshape(b, seq, d)


# speedup vs baseline: 1.1634x; 1.0014x over previous
import functools

import jax
import jax.numpy as jnp
from jax import lax
from jax.experimental import pallas as pl
from jax.experimental.pallas import tpu as pltpu
from jax.experimental.pallas import tpu_sc as plsc

T = 2048
D = 768
E = 8
H = 1536
B = 256
S = 6144
NB = S // B
NBP = 128

_f32 = jnp.float32
_i32 = jnp.int32


def _router_body(hs_ref, rw_ref, pos_ref, w0_ref, w1_ref, be_ref, nb_ref):
    hs = hs_ref[...]
    rw = rw_ref[...]
    st = lax.dot_general(rw, hs, (((1,), (1,)), ((), ())),
                         preferred_element_type=_f32)
    eio = lax.broadcasted_iota(_i32, (E, T), 0)
    m0 = jnp.max(st, axis=0, keepdims=True)
    idx0 = jnp.min(jnp.where(st == m0, eio, E), axis=0, keepdims=True)
    neg = jnp.where(eio == idx0, -jnp.inf, st)
    m1 = jnp.max(neg, axis=0, keepdims=True)
    idx1 = jnp.min(jnp.where(neg == m1, eio, E), axis=0, keepdims=True)

    sn = lax.dot_general(hs, rw, (((1,), (1,)), ((), ())),
                         preferred_element_type=_f32)
    lio = lax.broadcasted_iota(_i32, (T, E), 1)
    m0n = jnp.max(sn, axis=1, keepdims=True)
    i0n = jnp.min(jnp.where(sn == m0n, lio, E), axis=1, keepdims=True)
    m1n = jnp.max(jnp.where(lio == i0n, -jnp.inf, sn), axis=1, keepdims=True)
    sx = jnp.exp(m1n - m0n)
    w0_ref[...] = jnp.broadcast_to(1.0 / (1.0 + sx), (T, 16))
    w1_ref[...] = jnp.broadcast_to(sx / (1.0 + sx), (T, 16))

    erow = jnp.concatenate([idx0, idx1], axis=1)
    pio = lax.broadcasted_iota(_i32, (E, 2 * T), 0)
    p2 = (pio == erow).astype(_f32)
    counts = jnp.sum(p2, axis=1, keepdims=True)

    lane = lax.broadcasted_iota(_i32, (E, 2 * T), 1)
    c2 = p2
    sh = 1
    while sh < 2 * T:
        c2 = c2 + jnp.where(lane >= sh, pltpu.roll(c2, sh, 1), 0.0)
        sh *= 2
    rank = jnp.sum(p2 * c2, axis=0, keepdims=True) - 1.0

    pc = jnp.floor((counts + (B - 1)) / B) * B
    ts = (lax.broadcasted_iota(_i32, (E, E), 0)
          > lax.broadcasted_iota(_i32, (E, E), 1)).astype(_f32)
    offm = lax.dot_general(ts, jnp.broadcast_to(pc, (E, NBP)),
                           (((1,), (0,)), ((), ())),
                           preferred_element_type=_f32)
    off_pad = offm[:, :1]
    off_tok = jnp.sum(p2 * off_pad, axis=0, keepdims=True)
    pos_ref[...] = jnp.reshape(off_tok + rank, (2 * T,)).astype(_i32)

    nblk = pc / B
    cume = lax.dot_general(ts, jnp.broadcast_to(nblk, (E, NBP)),
                           (((1,), (0,)), ((), ())),
                           preferred_element_type=_f32)
    bio = lax.broadcasted_iota(_i32, (E, NBP), 1).astype(_f32)
    be = jnp.sum((bio >= cume).astype(_f32), axis=0) - 1.0
    be_ref[...] = be.astype(_i32)
    nb_ref[...] = jnp.broadcast_to(jnp.sum(nblk), (8,)).astype(_i32)


_router = pl.pallas_call(
    _router_body,
    out_shape=[
        jax.ShapeDtypeStruct((2 * T,), _i32),
        jax.ShapeDtypeStruct((T, 16), _f32),
        jax.ShapeDtypeStruct((T, 16), _f32),
        jax.ShapeDtypeStruct((NBP,), _i32),
        jax.ShapeDtypeStruct((8,), _i32),
    ],
)


def _dispatch_body(x_hbm, pos_hbm, xs_hbm, idx_v, rows_v, sem):
    c = lax.axis_index("c")
    s = lax.axis_index("s")
    pltpu.sync_copy(pos_hbm.at[pl.ds(c * T + s * 128, 128)], idx_v)
    pltpu.sync_copy(x_hbm.at[pl.ds(s * 128, 128)], rows_v)
    pltpu.async_copy(rows_v, xs_hbm.at[idx_v], sem).wait()


@functools.cache
def _make_dispatch():
    return pl.kernel(
        _dispatch_body,
        out_type=jax.ShapeDtypeStruct((S, D), _f32),
        mesh=plsc.VectorSubcoreMesh(core_axis_name="c", subcore_axis_name="s"),
        scratch_types=[
            pltpu.VMEM((128,), _i32),
            pltpu.VMEM((128, D), _f32),
            pltpu.SemaphoreType.DMA,
        ],
    )


NBMAX = S // B - 1


def _ffn_body(be_s, nb_s, xs_h, wg_h, wu_h, wd_h, ys_h,
              xv, gv, dv, yv, sx0, sx1, swA, swB, swC, sy0, sy1):
    sx = (sx0, sx1)
    sy = (sy0, sy1)
    sw = (swA, swB, swC)
    nbocc = nb_s[0]

    def start_x(b, par):
        pltpu.make_async_copy(xs_h.at[pl.ds(b * B, B)], xv.at[par],
                              sx[par]).start()

    def start_w(e, widx, cond):
        for wsel in range(3):
            @pl.when(cond & (widx == wsel))
            def _(wsel=wsel):
                pltpu.make_async_copy(wg_h.at[e], gv.at[wsel, pl.ds(0, H)],
                                      sw[wsel]).start()
                pltpu.make_async_copy(wu_h.at[e], gv.at[wsel, pl.ds(H, H)],
                                      sw[wsel]).start()
                pltpu.make_async_copy(wd_h.at[e], dv.at[wsel], sw[wsel]).start()

    def wait_w(widx, cond):
        for wsel in range(3):
            @pl.when(cond & (widx == wsel))
            def _(wsel=wsel):
                pltpu.make_async_copy(wg_h.at[0], gv.at[wsel, pl.ds(0, H)],
                                      sw[wsel]).wait()
                pltpu.make_async_copy(wu_h.at[0], gv.at[wsel, pl.ds(H, H)],
                                      sw[wsel]).wait()
                pltpu.make_async_copy(wd_h.at[0], dv.at[wsel], sw[wsel]).wait()

    true_ = jnp.bool_(True)
    e0 = be_s[0]
    e1p = be_s[1]
    start_x(0, 0)
    start_w(e0, jnp.int32(0), true_)
    new1p = (1 < nbocc) & (e1p != e0)
    start_w(e1p, jnp.int32(1), new1p)
    widx = jnp.int32(0)
    cur_e = e0
    fresh = true_

    for b in range(NBMAX):
        par = b % 2
        live = b < nbocc
        has_next = (b + 1) < nbocc
        e1 = be_s[min(b + 1, NBMAX - 1)]
        e2 = be_s[min(b + 2, NBMAX - 1)]
        new1 = has_next & (e1 != cur_e)
        new2 = ((b + 2) < nbocc) & (e2 != e1)
        w1 = (widx + new1.astype(_i32)) % 3
        w2 = (w1 + new2.astype(_i32)) % 3

        @pl.when(has_next)
        def _(b=b):
            start_x(b + 1, (b + 1) % 2)

        start_w(e2, w2, new2)

        @pl.when(live)
        def _():
            pltpu.make_async_copy(xs_h.at[pl.ds(0, B)], xv.at[par],
                                  sx[par]).wait()

        wait_w(widx, live & fresh)

        if b >= 2:
            @pl.when(live)
            def _():
                pltpu.make_async_copy(yv.at[par], ys_h.at[pl.ds(0, B)],
                                      sy[par]).wait()

        @pl.when(live)
        def _():
            x = xv[par]
            gu = lax.dot_general(x, gv[widx], (((1,), (1,)), ((), ())),
                                 preferred_element_type=_f32)
            g = gu[:, :H]
            u = gu[:, H:]
            h = g * lax.logistic(g) * u
            yv[par] = lax.dot_general(h, dv[widx], (((1,), (1,)), ((), ())),
                                      preferred_element_type=_f32)

        @pl.when(live)
        def _(b=b):
            pltpu.make_async_copy(yv.at[par], ys_h.at[pl.ds(b * B, B)],
                                  sy[par]).start()

        widx, cur_e, fresh = w1, e1, new1

    for par in (0, 1):
        pltpu.make_async_copy(yv.at[par], ys_h.at[pl.ds(0, B)],
                              sy[par]).wait()


_ffn = pl.pallas_call(
    _ffn_body,
    in_specs=[
        pl.BlockSpec(memory_space=pltpu.SMEM),
        pl.BlockSpec(memory_space=pltpu.SMEM),
        pl.BlockSpec(memory_space=pl.ANY),
        pl.BlockSpec(memory_space=pl.ANY),
        pl.BlockSpec(memory_space=pl.ANY),
        pl.BlockSpec(memory_space=pl.ANY),
    ],
    out_specs=pl.BlockSpec(memory_space=pl.ANY),
    out_shape=jax.ShapeDtypeStruct((S, D), _f32),
    scratch_shapes=[
        pltpu.VMEM((2, B, D), _f32),
        pltpu.VMEM((3, 2 * H, D), _f32),
        pltpu.VMEM((3, D, H), _f32),
        pltpu.VMEM((2, B, D), _f32),
        pltpu.SemaphoreType.DMA,
        pltpu.SemaphoreType.DMA,
        pltpu.SemaphoreType.DMA,
        pltpu.SemaphoreType.DMA,
        pltpu.SemaphoreType.DMA,
        pltpu.SemaphoreType.DMA,
        pltpu.SemaphoreType.DMA,
    ],
)


def _combine_body(pos_hbm, w0_hbm, w1_hbm, ys_hbm, out_hbm,
                  i0_v, i1_v, w0_v, w1_v, z0_v, z1_v, s0, s1):
    c = lax.axis_index("c")
    s = lax.axis_index("s")
    tb = (c * 16 + s) * 64
    pltpu.sync_copy(pos_hbm.at[pl.ds(tb, 64)], i0_v)
    pltpu.sync_copy(pos_hbm.at[pl.ds(T + tb, 64)], i1_v)
    cp0 = pltpu.async_copy(ys_hbm.at[i0_v], z0_v, s0)
    cp1 = pltpu.async_copy(ys_hbm.at[i1_v], z1_v, s1)
    pltpu.sync_copy(w0_hbm.at[pl.ds(tb, 64)], w0_v)
    pltpu.sync_copy(w1_hbm.at[pl.ds(tb, 64)], w1_v)
    cp0.wait()
    cp1.wait()

    def row(r, _):
        a = w0_v[r, pl.ds(0, 16)]
        b = w1_v[r, pl.ds(0, 16)]
        for ch in range(D // 16):
            sl = pl.ds(16 * ch, 16)
            z0_v[r, sl] = a * z0_v[r, sl] + b * z1_v[r, sl]
        return 0

    lax.fori_loop(0, 64, row, 0)
    pltpu.sync_copy(z0_v, out_hbm.at[pl.ds(tb, 64)])


@functools.cache
def _make_combine():
    return pl.kernel(
        _combine_body,
        out_type=jax.ShapeDtypeStruct((T, D), _f32),
        mesh=plsc.VectorSubcoreMesh(core_axis_name="c", subcore_axis_name="s"),
        scratch_types=[
            pltpu.VMEM((64,), _i32),
            pltpu.VMEM((64,), _i32),
            pltpu.VMEM((64, 16), _f32),
            pltpu.VMEM((64, 16), _f32),
            pltpu.VMEM((64, D), _f32),
            pltpu.VMEM((64, D), _f32),
            pltpu.SemaphoreType.DMA,
            pltpu.SemaphoreType.DMA,
        ],
    )


def kernel(x, router_w, w_gate, w_up, w_down):
    b, seq, d = x.shape
    hs = x.reshape(T, D)
    pos, w0, w1, be, nb = _router(hs, router_w)
    xs = _make_dispatch()(hs, pos)
    ys = _ffn(be, nb, xs, w_gate, w_up, w_down)
    out = _make_combine()(pos, w0, w1, ys)
    return out.reshape(b, seq, d)


# speedup vs baseline: 1.1683x; 1.0042x over previous
import functools

import jax
import jax.numpy as jnp
from jax import lax
from jax.experimental import pallas as pl
from jax.experimental.pallas import tpu as pltpu
from jax.experimental.pallas import tpu_sc as plsc

T = 2048
D = 768
E = 8
H = 1536
B = 256
S = 6144
NB = S // B
NBP = 128

_f32 = jnp.float32
_i32 = jnp.int32


def _router_body(hs_ref, rw_ref, pos_ref, w0_ref, w1_ref, be_ref, nb_ref):
    hs = hs_ref[...]
    rw = rw_ref[...]
    st = lax.dot_general(rw, hs, (((1,), (1,)), ((), ())),
                         preferred_element_type=_f32)
    eio = lax.broadcasted_iota(_i32, (E, T), 0)
    m0 = jnp.max(st, axis=0, keepdims=True)
    idx0 = jnp.min(jnp.where(st == m0, eio, E), axis=0, keepdims=True)
    neg = jnp.where(eio == idx0, -jnp.inf, st)
    m1 = jnp.max(neg, axis=0, keepdims=True)
    idx1 = jnp.min(jnp.where(neg == m1, eio, E), axis=0, keepdims=True)

    sn = lax.dot_general(hs, rw, (((1,), (1,)), ((), ())),
                         preferred_element_type=_f32)
    lio = lax.broadcasted_iota(_i32, (T, E), 1)
    m0n = jnp.max(sn, axis=1, keepdims=True)
    i0n = jnp.min(jnp.where(sn == m0n, lio, E), axis=1, keepdims=True)
    m1n = jnp.max(jnp.where(lio == i0n, -jnp.inf, sn), axis=1, keepdims=True)
    sx = jnp.exp(m1n - m0n)
    w0_ref[...] = jnp.broadcast_to(1.0 / (1.0 + sx), (T, 16))
    w1_ref[...] = jnp.broadcast_to(sx / (1.0 + sx), (T, 16))

    erow = jnp.concatenate([idx0, idx1], axis=1)
    pio = lax.broadcasted_iota(_i32, (E, 2 * T), 0)
    p2 = (pio == erow).astype(_f32)
    counts = jnp.sum(p2, axis=1, keepdims=True)

    lane = lax.broadcasted_iota(_i32, (E, 2 * T), 1)
    c2 = p2
    sh = 1
    while sh < 2 * T:
        c2 = c2 + jnp.where(lane >= sh, pltpu.roll(c2, sh, 1), 0.0)
        sh *= 2
    rank = jnp.sum(p2 * c2, axis=0, keepdims=True) - 1.0

    pc = jnp.floor((counts + (B - 1)) / B) * B
    ts = (lax.broadcasted_iota(_i32, (E, E), 0)
          > lax.broadcasted_iota(_i32, (E, E), 1)).astype(_f32)
    offm = lax.dot_general(ts, jnp.broadcast_to(pc, (E, NBP)),
                           (((1,), (0,)), ((), ())),
                           preferred_element_type=_f32)
    off_pad = offm[:, :1]
    off_tok = jnp.sum(p2 * off_pad, axis=0, keepdims=True)
    pos_ref[...] = jnp.reshape(off_tok + rank, (2 * T,)).astype(_i32)

    nblk = pc / B
    cume = lax.dot_general(ts, jnp.broadcast_to(nblk, (E, NBP)),
                           (((1,), (0,)), ((), ())),
                           preferred_element_type=_f32)
    bio = lax.broadcasted_iota(_i32, (E, NBP), 1).astype(_f32)
    be = jnp.sum((bio >= cume).astype(_f32), axis=0) - 1.0
    be_ref[...] = be.astype(_i32)
    nb_ref[...] = jnp.broadcast_to(jnp.sum(nblk), (8,)).astype(_i32)


_router = pl.pallas_call(
    _router_body,
    out_shape=[
        jax.ShapeDtypeStruct((2 * T,), _i32),
        jax.ShapeDtypeStruct((T, 16), _f32),
        jax.ShapeDtypeStruct((T, 16), _f32),
        jax.ShapeDtypeStruct((NBP,), _i32),
        jax.ShapeDtypeStruct((8,), _i32),
    ],
)


def _dispatch_body(x_hbm, pos_hbm, xs_hbm, idx_v, rows_v, sem):
    c = lax.axis_index("c")
    s = lax.axis_index("s")
    pltpu.sync_copy(pos_hbm.at[pl.ds(c * T + s * 128, 128)], idx_v)
    pltpu.sync_copy(x_hbm.at[pl.ds(s * 128, 128)], rows_v)
    pltpu.async_copy(rows_v, xs_hbm.at[idx_v], sem).wait()


@functools.cache
def _make_dispatch():
    return pl.kernel(
        _dispatch_body,
        out_type=jax.ShapeDtypeStruct((S, D), _f32),
        mesh=plsc.VectorSubcoreMesh(core_axis_name="c", subcore_axis_name="s"),
        scratch_types=[
            pltpu.VMEM((128,), _i32),
            pltpu.VMEM((128, D), _f32),
            pltpu.SemaphoreType.DMA,
        ],
    )


NBMAX = S // B - 1


def _ffn_body(be_s, nb_s, xs_h, wg_h, wu_h, wd_h, ys_h,
              xv, gv, dv, yv, sx0, sx1, swA, swB, swC, sy0, sy1):
    sx = (sx0, sx1)
    sy = (sy0, sy1)
    sw = (swA, swB, swC)
    nbocc = nb_s[0]

    def start_x(b, par):
        pltpu.make_async_copy(xs_h.at[pl.ds(b * B, B)], xv.at[par],
                              sx[par]).start()

    def start_w(e, widx, cond):
        for wsel in range(3):
            @pl.when(cond & (widx == wsel))
            def _(wsel=wsel):
                pltpu.make_async_copy(wg_h.at[e], gv.at[wsel, pl.ds(0, H)],
                                      sw[wsel]).start()
                pltpu.make_async_copy(wu_h.at[e], gv.at[wsel, pl.ds(H, H)],
                                      sw[wsel]).start()
                pltpu.make_async_copy(wd_h.at[e], dv.at[wsel], sw[wsel]).start()

    def wait_w(widx, cond):
        for wsel in range(3):
            @pl.when(cond & (widx == wsel))
            def _(wsel=wsel):
                pltpu.make_async_copy(wg_h.at[0], gv.at[wsel, pl.ds(0, H)],
                                      sw[wsel]).wait()
                pltpu.make_async_copy(wu_h.at[0], gv.at[wsel, pl.ds(H, H)],
                                      sw[wsel]).wait()
                pltpu.make_async_copy(wd_h.at[0], dv.at[wsel], sw[wsel]).wait()

    true_ = jnp.bool_(True)
    e0 = be_s[0]
    e1p = be_s[1]
    start_x(0, 0)
    start_w(e0, jnp.int32(0), true_)
    new1p = (1 < nbocc) & (e1p != e0)
    start_w(e1p, jnp.int32(1), new1p)
    widx = jnp.int32(0)
    cur_e = e0
    fresh = true_

    for b in range(NBMAX):
        par = b % 2
        live = b < nbocc
        has_next = (b + 1) < nbocc
        e1 = be_s[min(b + 1, NBMAX - 1)]
        e2 = be_s[min(b + 2, NBMAX - 1)]
        new1 = has_next & (e1 != cur_e)
        new2 = ((b + 2) < nbocc) & (e2 != e1)
        w1 = (widx + new1.astype(_i32)) % 3
        w2 = (w1 + new2.astype(_i32)) % 3

        @pl.when(has_next)
        def _(b=b):
            start_x(b + 1, (b + 1) % 2)

        start_w(e2, w2, new2)

        @pl.when(live)
        def _():
            pltpu.make_async_copy(xs_h.at[pl.ds(0, B)], xv.at[par],
                                  sx[par]).wait()

        wait_w(widx, live & fresh)

        if b >= 2:
            @pl.when(live)
            def _():
                pltpu.make_async_copy(yv.at[par], ys_h.at[pl.ds(0, B)],
                                      sy[par]).wait()

        @pl.when(live)
        def _():
            x = xv[par].astype(jnp.bfloat16)
            gu = lax.dot_general(x, gv[widx].astype(jnp.bfloat16),
                                 (((1,), (1,)), ((), ())),
                                 preferred_element_type=_f32)
            g = gu[:, :H]
            u = gu[:, H:]
            h = (g * lax.logistic(g) * u).astype(jnp.bfloat16)
            yv[par] = lax.dot_general(h, dv[widx].astype(jnp.bfloat16),
                                      (((1,), (1,)), ((), ())),
                                      preferred_element_type=_f32)

        @pl.when(live)
        def _(b=b):
            pltpu.make_async_copy(yv.at[par], ys_h.at[pl.ds(b * B, B)],
                                  sy[par]).start()

        widx, cur_e, fresh = w1, e1, new1

    for par in (0, 1):
        pltpu.make_async_copy(yv.at[par], ys_h.at[pl.ds(0, B)],
                              sy[par]).wait()


_ffn = pl.pallas_call(
    _ffn_body,
    in_specs=[
        pl.BlockSpec(memory_space=pltpu.SMEM),
        pl.BlockSpec(memory_space=pltpu.SMEM),
        pl.BlockSpec(memory_space=pl.ANY),
        pl.BlockSpec(memory_space=pl.ANY),
        pl.BlockSpec(memory_space=pl.ANY),
        pl.BlockSpec(memory_space=pl.ANY),
    ],
    out_specs=pl.BlockSpec(memory_space=pl.ANY),
    out_shape=jax.ShapeDtypeStruct((S, D), _f32),
    scratch_shapes=[
        pltpu.VMEM((2, B, D), _f32),
        pltpu.VMEM((3, 2 * H, D), _f32),
        pltpu.VMEM((3, D, H), _f32),
        pltpu.VMEM((2, B, D), _f32),
        pltpu.SemaphoreType.DMA,
        pltpu.SemaphoreType.DMA,
        pltpu.SemaphoreType.DMA,
        pltpu.SemaphoreType.DMA,
        pltpu.SemaphoreType.DMA,
        pltpu.SemaphoreType.DMA,
        pltpu.SemaphoreType.DMA,
    ],
)


def _combine_body(pos_hbm, w0_hbm, w1_hbm, ys_hbm, out_hbm,
                  i0_v, i1_v, w0_v, w1_v, z0_v, z1_v, s0, s1):
    c = lax.axis_index("c")
    s = lax.axis_index("s")
    tb = (c * 16 + s) * 64
    pltpu.sync_copy(pos_hbm.at[pl.ds(tb, 64)], i0_v)
    pltpu.sync_copy(pos_hbm.at[pl.ds(T + tb, 64)], i1_v)
    cp0 = pltpu.async_copy(ys_hbm.at[i0_v], z0_v, s0)
    cp1 = pltpu.async_copy(ys_hbm.at[i1_v], z1_v, s1)
    pltpu.sync_copy(w0_hbm.at[pl.ds(tb, 64)], w0_v)
    pltpu.sync_copy(w1_hbm.at[pl.ds(tb, 64)], w1_v)
    cp0.wait()
    cp1.wait()

    def row(r, _):
        a = w0_v[r, pl.ds(0, 16)]
        b = w1_v[r, pl.ds(0, 16)]
        for ch in range(D // 16):
            sl = pl.ds(16 * ch, 16)
            z0_v[r, sl] = a * z0_v[r, sl] + b * z1_v[r, sl]
        return 0

    lax.fori_loop(0, 64, row, 0)
    pltpu.sync_copy(z0_v, out_hbm.at[pl.ds(tb, 64)])


@functools.cache
def _make_combine():
    return pl.kernel(
        _combine_body,
        out_type=jax.ShapeDtypeStruct((T, D), _f32),
        mesh=plsc.VectorSubcoreMesh(core_axis_name="c", subcore_axis_name="s"),
        scratch_types=[
            pltpu.VMEM((64,), _i32),
            pltpu.VMEM((64,), _i32),
            pltpu.VMEM((64, 16), _f32),
            pltpu.VMEM((64, 16), _f32),
            pltpu.VMEM((64, D), _f32),
            pltpu.VMEM((64, D), _f32),
            pltpu.SemaphoreType.DMA,
            pltpu.SemaphoreType.DMA,
        ],
    )


def kernel(x, router_w, w_gate, w_up, w_down):
    b, seq, d = x.shape
    hs = x.reshape(T, D)
    pos, w0, w1, be, nb = _router(hs, router_w)
    xs = _make_dispatch()(hs, pos)
    ys = _ffn(be, nb, xs, w_gate, w_up, w_down)
    out = _make_combine()(pos, w0, w1, ys)
    return out.reshape(b, seq, d)
